# baseline (speedup 1.0000x reference)
import functools

import jax
import jax.numpy as jnp
from jax import lax
from jax.experimental import pallas as pl
from jax.experimental.pallas import tpu as pltpu
from jax.experimental.pallas import tpu_sc as plsc

N = 10000
E = 320000
G = 64
DIN = 128
DH = 64
DOUT = 10

NC = 2
NS = 16
L = 16
CHUNK = 128
NCH0 = 160
NPHASE = 2
NCHP = NCH0 // NPHASE
NCHD = 80
EPAD = NS * NCH0 * CHUNK
NPAD = 10112
STRIPE = NPAD // NS
NBUF = 5

_mesh = plsc.VectorSubcoreMesh(core_axis_name="c", subcore_axis_name="s")
_sc_params = pltpu.CompilerParams(use_tc_tiling_on_sc=False)


def _zero_vmem_2d(buf, rows, cols):
    z = jnp.zeros((L,), jnp.float32)

    @pl.loop(0, rows)
    def _(r):
        @pl.loop(0, cols, step=L)
        def _(j):
            buf[r, pl.ds(j, L)] = z


def _init_acc_from(zb, acc, base, rows):
    @pl.loop(0, rows // CHUNK)
    def _(k):
        pltpu.sync_copy(zb, acc.at[pl.ds(base + k * CHUNK, CHUNK)])

    rem = rows % CHUNK
    if rem:
        pltpu.sync_copy(
            zb.at[pl.ds(0, rem)],
            acc.at[pl.ds(base + (rows // CHUNK) * CHUNK, rem)],
        )


@functools.partial(
    pl.kernel,
    out_type=jax.ShapeDtypeStruct((NC, NPAD, L), jnp.float32),
    mesh=_mesh,
    compiler_params=_sc_params,
    scratch_types=[
        pltpu.VMEM((NCHD, CHUNK), jnp.int32),
        pltpu.VMEM((CHUNK, L), jnp.float32),
        pltpu.VMEM_SHARED((NPAD, L), jnp.float32),
        pltpu.SemaphoreType.DMA,
    ],
)
def _sc_degree(dst_hbm, out_hbm, dstv, ones_v, acc, ssem):
    cid = lax.axis_index("c")
    sid = lax.axis_index("s")
    nch = NCHD

    _zero_vmem_2d(ones_v, CHUNK, L)
    _init_acc_from(ones_v, acc, sid * STRIPE, STRIPE)

    one = jnp.ones((L,), jnp.float32)

    @pl.loop(0, CHUNK)
    def _(r):
        ones_v[r, pl.ds(0, L)] = one

    pltpu.sync_copy(dst_hbm.at[cid, sid], dstv)
    plsc.subcore_barrier()

    @pl.loop(0, nch)
    def _(c):
        pltpu.async_copy(ones_v, acc.at[dstv.at[c]], ssem, add=True)

    @pl.loop(0, nch)
    def _(c):
        pltpu.make_async_copy(ones_v, acc.at[dstv.at[c]], ssem).wait()

    plsc.subcore_barrier()
    sl = pl.ds(sid * STRIPE, STRIPE)
    pltpu.sync_copy(acc.at[sl], out_hbm.at[cid, sl])


@functools.partial(
    pl.kernel,
    out_type=jax.ShapeDtypeStruct((NPAD, DH), jnp.float32),
    mesh=_mesh,
    compiler_params=_sc_params,
    scratch_types=[
        pltpu.VMEM((NCHP, CHUNK), jnp.int32),
        pltpu.VMEM((NCHP, CHUNK), jnp.int32),
        [pltpu.VMEM((CHUNK, DH), jnp.float32)] * NBUF,
        pltpu.VMEM_SHARED((NPAD, DH), jnp.float32),
        [pltpu.SemaphoreType.DMA] * NBUF,
        [pltpu.SemaphoreType.DMA] * NBUF,
    ],
)
def _sc_scatter(hs_hbm, src_hbm, dst_hbm, out_hbm,
                srcv, dstv, bufs, acc, gsems, ssems):
    cid = lax.axis_index("c")
    sid = lax.axis_index("s")

    @pl.when(cid == 0)
    def _():
        _zero_vmem_2d(bufs[0], CHUNK, DH)
        _init_acc_from(bufs[0], acc, sid * STRIPE, STRIPE)
        plsc.subcore_barrier()

        def gather(c, b):
            pltpu.async_copy(hs_hbm.at[srcv.at[c]], bufs[b], gsems[b])

        def wait_gather(c, b):
            pltpu.make_async_copy(
                hs_hbm.at[srcv.at[c]], bufs[b], gsems[b]).wait()

        def scatter(c, b):
            pltpu.async_copy(bufs[b], acc.at[dstv.at[c]], ssems[b], add=True)

        def wait_scatter(c, b):
            pltpu.make_async_copy(
                bufs[b], acc.at[dstv.at[c]], ssems[b]).wait()

        for p in range(NPHASE):
            pltpu.sync_copy(src_hbm.at[sid, pl.ds(p * NCHP, NCHP)], srcv)
            pltpu.sync_copy(dst_hbm.at[sid, pl.ds(p * NCHP, NCHP)], dstv)

            for b in range(NBUF):
                gather(b, b)

            @pl.loop(0, NCHP, step=NBUF)
            def _(c):
                for b in range(NBUF):
                    wait_gather(c + b, b)
                    scatter(c + b, b)
                for b in range(NBUF):
                    wait_scatter(c + b, b)

                    @pl.when(c + NBUF + b < NCHP)
                    def _():
                        gather(c + NBUF + b, b)

        plsc.subcore_barrier()
        sl = pl.ds(sid * STRIPE, STRIPE)
        pltpu.sync_copy(acc.at[sl], out_hbm.at[sl])


def _dinv_col(degp):
    deg = degp[0] + degp[1]
    dinv = jnp.where(deg > 0.0, lax.rsqrt(jnp.maximum(deg, 1e-12)), 0.0)
    return lax.slice(dinv, (0, 0), (NPAD, 1))


def _tc1_body(x_ref, w1_ref, degp_ref, hs_ref):
    dinv = _dinv_col(degp_ref[...])
    xw = jnp.dot(x_ref[...], w1_ref[...], preferred_element_type=jnp.float32)
    hs_ref[...] = xw * dinv


def _tc2_body(agg_ref, degp_ref, b1_ref, w2_ref, hs_ref):
    dinv = _dinv_col(degp_ref[...])
    h1 = jax.nn.relu(dinv * agg_ref[...] + b1_ref[...])
    hw = jnp.dot(h1, w2_ref[...], preferred_element_type=jnp.float32)
    hs_ref[...] = hw * dinv


def _tc3_body(agg_ref, degp_ref, b2_ref, batch_ref, wfc_ref, bfc_ref, out_ref):
    dinv = _dinv_col(degp_ref[...])
    h2 = jax.nn.relu(dinv * agg_ref[...] + b2_ref[...])
    gids = lax.broadcasted_iota(jnp.int32, (G, NPAD), 0)
    onehot_t = (gids == batch_ref[...]).astype(jnp.float32)
    g = jnp.dot(onehot_t, h2, preferred_element_type=jnp.float32)
    logits = jnp.dot(g, wfc_ref[...], preferred_element_type=jnp.float32)
    logits = logits + bfc_ref[...]
    m = jnp.max(logits, axis=1, keepdims=True)
    z = logits - m
    lse = jnp.log(jnp.sum(jnp.exp(z), axis=1, keepdims=True))
    out_ref[...] = z - lse


_tc1 = pl.pallas_call(
    _tc1_body, out_shape=jax.ShapeDtypeStruct((NPAD, DH), jnp.float32))
_tc2 = pl.pallas_call(
    _tc2_body, out_shape=jax.ShapeDtypeStruct((NPAD, DH), jnp.float32))
_tc3 = pl.pallas_call(
    _tc3_body, out_shape=jax.ShapeDtypeStruct((G, DOUT), jnp.float32))


def _chunked(idx, shape):
    pad = jnp.full((EPAD - E,), N, jnp.int32)
    return jnp.concatenate([idx, pad]).reshape(shape)


def kernel(x, edge_index, batch, W1, b1, W2, b2, Wfc, bfc):
    x_pad = jnp.zeros((NPAD, DIN), jnp.float32).at[:N].set(x)
    src3 = _chunked(edge_index[0], (NS, NCH0, CHUNK))
    dst3 = _chunked(edge_index[1], (NS, NCH0, CHUNK))
    dstd = _chunked(edge_index[1], (NC, NS, NCHD, CHUNK))
    batch2 = jnp.concatenate(
        [batch.astype(jnp.int32), jnp.full((NPAD - N,), G, jnp.int32)]
    ).reshape(1, NPAD)

    degp = _sc_degree(dstd)
    hs1 = _tc1(x_pad, W1, degp)
    agg1 = _sc_scatter(hs1, src3, dst3)
    hs2 = _tc2(agg1, degp, b1.reshape(1, DH), W2)
    agg2 = _sc_scatter(hs2, src3, dst3)
    return _tc3(agg2, degp, b2.reshape(1, DH), batch2, Wfc, bfc.reshape(1, DOUT))


# speedup vs baseline: 2.1246x; 2.1246x over previous
import functools

import jax
import jax.numpy as jnp
from jax import lax
from jax.experimental import pallas as pl
from jax.experimental.pallas import tpu as pltpu
from jax.experimental.pallas import tpu_sc as plsc

N = 10000
E = 320000
G = 64
DIN = 128
DH = 64
DOUT = 10

NC = 2
NS = 16
L = 16
CHUNK = 128
NCH0 = 160
NPHASE = 2
NCHP = NCH0 // NPHASE
NCHD = 80
EPAD = NS * NCH0 * CHUNK
NPAD = 10112
STRIPE = NPAD // NS
NBUF = 5

_mesh = plsc.VectorSubcoreMesh(core_axis_name="c", subcore_axis_name="s")
_sc_params = pltpu.CompilerParams(use_tc_tiling_on_sc=False)


def _zero_vmem_2d(buf, rows, cols):
    z = jnp.zeros((L,), jnp.float32)

    @pl.loop(0, rows)
    def _(r):
        @pl.loop(0, cols, step=L)
        def _(j):
            buf[r, pl.ds(j, L)] = z


def _init_acc_from(zb, acc, base, rows):
    @pl.loop(0, rows // CHUNK)
    def _(k):
        pltpu.sync_copy(zb, acc.at[pl.ds(base + k * CHUNK, CHUNK)])

    rem = rows % CHUNK
    if rem:
        pltpu.sync_copy(
            zb.at[pl.ds(0, rem)],
            acc.at[pl.ds(base + (rows // CHUNK) * CHUNK, rem)],
        )


@functools.partial(
    pl.kernel,
    out_type=jax.ShapeDtypeStruct((NC, NPAD, L), jnp.float32),
    mesh=_mesh,
    compiler_params=_sc_params,
    scratch_types=[
        pltpu.VMEM((NCHD, CHUNK), jnp.int32),
        pltpu.VMEM((CHUNK, L), jnp.float32),
        pltpu.VMEM_SHARED((NPAD, L), jnp.float32),
        pltpu.SemaphoreType.DMA,
    ],
)
def _sc_degree(dst_hbm, out_hbm, dstv, ones_v, acc, ssem):
    cid = lax.axis_index("c")
    sid = lax.axis_index("s")
    nch = NCHD

    _zero_vmem_2d(ones_v, CHUNK, L)
    _init_acc_from(ones_v, acc, sid * STRIPE, STRIPE)

    one = jnp.ones((L,), jnp.float32)

    @pl.loop(0, CHUNK)
    def _(r):
        ones_v[r, pl.ds(0, L)] = one

    pltpu.sync_copy(dst_hbm.at[cid, sid], dstv)
    plsc.subcore_barrier()

    @pl.loop(0, nch)
    def _(c):
        pltpu.async_copy(ones_v, acc.at[dstv.at[c]], ssem, add=True)

    @pl.loop(0, nch)
    def _(c):
        pltpu.make_async_copy(ones_v, acc.at[dstv.at[c]], ssem).wait()

    plsc.subcore_barrier()
    sl = pl.ds(sid * STRIPE, STRIPE)
    pltpu.sync_copy(acc.at[sl], out_hbm.at[cid, sl])


@functools.partial(
    pl.kernel,
    out_type=jax.ShapeDtypeStruct((NPAD, DH), jnp.float32),
    mesh=_mesh,
    compiler_params=_sc_params,
    scratch_types=[
        pltpu.VMEM((NCHP, CHUNK), jnp.int32),
        pltpu.VMEM((NCHP, CHUNK), jnp.int32),
        [pltpu.VMEM((CHUNK, DH), jnp.float32)] * NBUF,
        pltpu.VMEM_SHARED((NPAD, DH), jnp.float32),
        [pltpu.SemaphoreType.DMA] * NBUF,
        [pltpu.SemaphoreType.DMA] * NBUF,
    ],
)
def _sc_scatter(hs_hbm, src_hbm, dst_hbm, out_hbm,
                srcv, dstv, bufs, acc, gsems, ssems):
    cid = lax.axis_index("c")
    sid = lax.axis_index("s")

    @pl.when(cid == 0)
    def _():
        _zero_vmem_2d(bufs[0], CHUNK, DH)
        _init_acc_from(bufs[0], acc, sid * STRIPE, STRIPE)
        plsc.subcore_barrier()

        def gather(c, b):
            pltpu.async_copy(hs_hbm.at[srcv.at[c]], bufs[b], gsems[b])

        def wait_gather(c, b):
            pltpu.make_async_copy(
                hs_hbm.at[srcv.at[c]], bufs[b], gsems[b]).wait()

        def scatter(c, b):
            pltpu.async_copy(bufs[b], acc.at[dstv.at[c]], ssems[b], add=True)

        def wait_scatter(c, b):
            pltpu.make_async_copy(
                bufs[b], acc.at[dstv.at[c]], ssems[b]).wait()

        for p in range(NPHASE):
            pltpu.sync_copy(src_hbm.at[sid, pl.ds(p * NCHP, NCHP)], srcv)
            pltpu.sync_copy(dst_hbm.at[sid, pl.ds(p * NCHP, NCHP)], dstv)

            for b in range(NBUF):
                gather(b, b)

            @pl.loop(0, NCHP, step=NBUF)
            def _(c):
                for b in range(NBUF):
                    wait_gather(c + b, b)
                    scatter(c + b, b)
                for b in range(NBUF):
                    wait_scatter(c + b, b)

                    @pl.when(c + NBUF + b < NCHP)
                    def _():
                        gather(c + NBUF + b, b)

        plsc.subcore_barrier()
        sl = pl.ds(sid * STRIPE, STRIPE)
        pltpu.sync_copy(acc.at[sl], out_hbm.at[sl])


def _dinv_col(degp):
    deg = degp[0] + degp[1]
    dinv = jnp.where(deg > 0.0, lax.rsqrt(jnp.maximum(deg, 1e-12)), 0.0)
    return lax.slice(dinv, (0, 0), (NPAD, 1))


def _tc1_body(x_ref, w1_ref, degp_ref, hs_ref):
    dinv = _dinv_col(degp_ref[...])
    xw = jnp.dot(x_ref[...], w1_ref[...], preferred_element_type=jnp.float32)
    hs_ref[...] = xw * dinv


def _tc2_body(agg_ref, degp_ref, b1_ref, w2_ref, hs_ref):
    dinv = _dinv_col(degp_ref[...])
    h1 = jax.nn.relu(dinv * agg_ref[...] + b1_ref[...])
    hw = jnp.dot(h1, w2_ref[...], preferred_element_type=jnp.float32)
    hs_ref[...] = hw * dinv


def _tc3_body(agg_ref, degp_ref, b2_ref, batch_ref, wfc_ref, bfc_ref, out_ref):
    dinv = _dinv_col(degp_ref[...])
    h2 = jax.nn.relu(dinv * agg_ref[...] + b2_ref[...])
    gids = lax.broadcasted_iota(jnp.int32, (G, NPAD), 0)
    onehot_t = (gids == batch_ref[...]).astype(jnp.float32)
    g = jnp.dot(onehot_t, h2, preferred_element_type=jnp.float32)
    logits = jnp.dot(g, wfc_ref[...], preferred_element_type=jnp.float32)
    logits = logits + bfc_ref[...]
    m = jnp.max(logits, axis=1, keepdims=True)
    z = logits - m
    lse = jnp.log(jnp.sum(jnp.exp(z), axis=1, keepdims=True))
    out_ref[...] = z - lse


_tc1 = pl.pallas_call(
    _tc1_body, out_shape=jax.ShapeDtypeStruct((NPAD, DH), jnp.float32))
_tc2 = pl.pallas_call(
    _tc2_body, out_shape=jax.ShapeDtypeStruct((NPAD, DH), jnp.float32))
_tc3 = pl.pallas_call(
    _tc3_body, out_shape=jax.ShapeDtypeStruct((G, DOUT), jnp.float32))


def _chunked(idx, shape):
    pad = N + (jnp.arange(EPAD - E, dtype=jnp.int32) % (NPAD - N))
    return jnp.concatenate([idx, pad]).reshape(shape)


def kernel(x, edge_index, batch, W1, b1, W2, b2, Wfc, bfc):
    x_pad = jnp.zeros((NPAD, DIN), jnp.float32).at[:N].set(x)
    src3 = _chunked(edge_index[0], (NS, NCH0, CHUNK))
    dst3 = _chunked(edge_index[1], (NS, NCH0, CHUNK))
    dstd = _chunked(edge_index[1], (NC, NS, NCHD, CHUNK))
    batch2 = jnp.concatenate(
        [batch.astype(jnp.int32), jnp.full((NPAD - N,), G, jnp.int32)]
    ).reshape(1, NPAD)

    degp = _sc_degree(dstd)
    hs1 = _tc1(x_pad, W1, degp)
    agg1 = _sc_scatter(hs1, src3, dst3)
    hs2 = _tc2(agg1, degp, b1.reshape(1, DH), W2)
    agg2 = _sc_scatter(hs2, src3, dst3)
    return _tc3(agg2, degp, b2.reshape(1, DH), batch2, Wfc, bfc.reshape(1, DOUT))


# speedup vs baseline: 2.9037x; 1.3667x over previous
import functools

import jax
import jax.numpy as jnp
from jax import lax
from jax.experimental import pallas as pl
from jax.experimental.pallas import tpu as pltpu
from jax.experimental.pallas import tpu_sc as plsc

N = 10000
E = 320000
G = 64
DIN = 128
DH = 64
DOUT = 10

NC = 2
NS = 16
L = 16
CHUNK = 128
NCHW = 80
NCH0 = NC * NCHW
NCHD = 80
EPAD = NC * NS * NCHW * CHUNK
NPAD = 10112
STRIPE = NPAD // NS
NBUF = 5

_mesh = plsc.VectorSubcoreMesh(core_axis_name="c", subcore_axis_name="s")
_sc_params = pltpu.CompilerParams(use_tc_tiling_on_sc=False)


def _zero_vmem_2d(buf, rows, cols):
    z = jnp.zeros((L,), jnp.float32)

    @pl.loop(0, rows)
    def _(r):
        @pl.loop(0, cols, step=L)
        def _(j):
            buf[r, pl.ds(j, L)] = z


def _init_acc_from(zb, acc, base, rows):
    @pl.loop(0, rows // CHUNK)
    def _(k):
        pltpu.sync_copy(zb, acc.at[pl.ds(base + k * CHUNK, CHUNK)])

    rem = rows % CHUNK
    if rem:
        pltpu.sync_copy(
            zb.at[pl.ds(0, rem)],
            acc.at[pl.ds(base + (rows // CHUNK) * CHUNK, rem)],
        )


@functools.partial(
    pl.kernel,
    out_type=jax.ShapeDtypeStruct((NC, NPAD, L), jnp.float32),
    mesh=_mesh,
    compiler_params=_sc_params,
    scratch_types=[
        pltpu.VMEM((NCHD, CHUNK), jnp.int32),
        pltpu.VMEM((CHUNK, L), jnp.float32),
        pltpu.VMEM_SHARED((NPAD, L), jnp.float32),
        pltpu.SemaphoreType.DMA,
    ],
)
def _sc_degree(dst_hbm, out_hbm, dstv, ones_v, acc, ssem):
    cid = lax.axis_index("c")
    sid = lax.axis_index("s")
    nch = NCHD

    _zero_vmem_2d(ones_v, CHUNK, L)
    _init_acc_from(ones_v, acc, sid * STRIPE, STRIPE)

    one = jnp.ones((L,), jnp.float32)

    @pl.loop(0, CHUNK)
    def _(r):
        ones_v[r, pl.ds(0, L)] = one

    pltpu.sync_copy(dst_hbm.at[cid, sid], dstv)
    plsc.subcore_barrier()

    @pl.loop(0, nch)
    def _(c):
        pltpu.async_copy(ones_v, acc.at[dstv.at[c]], ssem, add=True)

    @pl.loop(0, nch)
    def _(c):
        pltpu.make_async_copy(ones_v, acc.at[dstv.at[c]], ssem).wait()

    plsc.subcore_barrier()
    sl = pl.ds(sid * STRIPE, STRIPE)
    pltpu.sync_copy(acc.at[sl], out_hbm.at[cid, sl])


@functools.partial(
    pl.kernel,
    out_type=jax.ShapeDtypeStruct((NC, NPAD, DH), jnp.float32),
    mesh=_mesh,
    compiler_params=_sc_params,
    scratch_types=[
        pltpu.VMEM((NCHW, CHUNK), jnp.int32),
        pltpu.VMEM((NCHW, CHUNK), jnp.int32),
        [pltpu.VMEM((CHUNK, DH), jnp.float32)] * NBUF,
        pltpu.VMEM_SHARED((NPAD, DH), jnp.float32),
        [pltpu.SemaphoreType.DMA] * NBUF,
        [pltpu.SemaphoreType.DMA] * NBUF,
    ],
)
def _sc_scatter(hs_hbm, src_hbm, dst_hbm, out_hbm,
                srcv, dstv, bufs, acc, gsems, ssems):
    cid = lax.axis_index("c")
    sid = lax.axis_index("s")

    _zero_vmem_2d(bufs[0], CHUNK, DH)
    _init_acc_from(bufs[0], acc, sid * STRIPE, STRIPE)
    pltpu.sync_copy(src_hbm.at[cid, sid], srcv)
    pltpu.sync_copy(dst_hbm.at[cid, sid], dstv)
    plsc.subcore_barrier()

    def gather(c, b):
        pltpu.async_copy(hs_hbm.at[srcv.at[c]], bufs[b], gsems[b])

    def wait_gather(c, b):
        pltpu.make_async_copy(hs_hbm.at[srcv.at[c]], bufs[b], gsems[b]).wait()

    def scatter(c, b):
        pltpu.async_copy(bufs[b], acc.at[dstv.at[c]], ssems[b], add=True)

    def wait_scatter(c, b):
        pltpu.make_async_copy(bufs[b], acc.at[dstv.at[c]], ssems[b]).wait()

    for b in range(NBUF):
        gather(b, b)

    @pl.loop(0, NCHW, step=NBUF)
    def _(c):
        for b in range(NBUF):
            wait_gather(c + b, b)
            scatter(c + b, b)
        for b in range(NBUF):
            wait_scatter(c + b, b)

            @pl.when(c + NBUF + b < NCHW)
            def _():
                gather(c + NBUF + b, b)

    plsc.subcore_barrier()
    sl = pl.ds(sid * STRIPE, STRIPE)
    pltpu.sync_copy(acc.at[sl], out_hbm.at[cid, sl])


def _dinv_col(degp):
    deg = degp[0] + degp[1]
    dinv = jnp.where(deg > 0.0, lax.rsqrt(jnp.maximum(deg, 1e-12)), 0.0)
    return lax.slice(dinv, (0, 0), (NPAD, 1))


def _tc1_body(x_ref, w1_ref, degp_ref, hs_ref):
    dinv = _dinv_col(degp_ref[...])
    xw = jnp.dot(x_ref[...], w1_ref[...], preferred_element_type=jnp.float32)
    hs_ref[...] = xw * dinv


def _tc2_body(agg_ref, degp_ref, b1_ref, w2_ref, hs_ref):
    dinv = _dinv_col(degp_ref[...])
    aggp = agg_ref[...]
    h1 = jax.nn.relu(dinv * (aggp[0] + aggp[1]) + b1_ref[...])
    hw = jnp.dot(h1, w2_ref[...], preferred_element_type=jnp.float32)
    hs_ref[...] = hw * dinv


def _tc3_body(agg_ref, degp_ref, b2_ref, batch_ref, wfc_ref, bfc_ref, out_ref):
    dinv = _dinv_col(degp_ref[...])
    aggp = agg_ref[...]
    h2 = jax.nn.relu(dinv * (aggp[0] + aggp[1]) + b2_ref[...])
    gids = lax.broadcasted_iota(jnp.int32, (G, NPAD), 0)
    onehot_t = (gids == batch_ref[...]).astype(jnp.float32)
    g = jnp.dot(onehot_t, h2, preferred_element_type=jnp.float32)
    logits = jnp.dot(g, wfc_ref[...], preferred_element_type=jnp.float32)
    logits = logits + bfc_ref[...]
    m = jnp.max(logits, axis=1, keepdims=True)
    z = logits - m
    lse = jnp.log(jnp.sum(jnp.exp(z), axis=1, keepdims=True))
    out_ref[...] = z - lse


_tc1 = pl.pallas_call(
    _tc1_body, out_shape=jax.ShapeDtypeStruct((NPAD, DH), jnp.float32))
_tc2 = pl.pallas_call(
    _tc2_body, out_shape=jax.ShapeDtypeStruct((NPAD, DH), jnp.float32))
_tc3 = pl.pallas_call(
    _tc3_body, out_shape=jax.ShapeDtypeStruct((G, DOUT), jnp.float32))


def _chunked(idx, shape):
    pad = N + (jnp.arange(EPAD - E, dtype=jnp.int32) % (NPAD - N))
    return jnp.concatenate([idx, pad]).reshape(shape)


def kernel(x, edge_index, batch, W1, b1, W2, b2, Wfc, bfc):
    x_pad = jnp.zeros((NPAD, DIN), jnp.float32).at[:N].set(x)
    src3 = _chunked(edge_index[0], (NC, NS, NCHW, CHUNK))
    dst3 = _chunked(edge_index[1], (NC, NS, NCHW, CHUNK))
    dstd = _chunked(edge_index[1], (NC, NS, NCHD, CHUNK))
    batch2 = jnp.concatenate(
        [batch.astype(jnp.int32), jnp.full((NPAD - N,), G, jnp.int32)]
    ).reshape(1, NPAD)

    degp = _sc_degree(dstd)
    hs1 = _tc1(x_pad, W1, degp)
    agg1 = _sc_scatter(hs1, src3, dst3)
    hs2 = _tc2(agg1, degp, b1.reshape(1, DH), W2)
    agg2 = _sc_scatter(hs2, src3, dst3)
    return _tc3(agg2, degp, b2.reshape(1, DH), batch2, Wfc, bfc.reshape(1, DOUT))
